# baseline (speedup 1.0000x reference)
import functools
import math

import jax
import jax.numpy as jnp
from jax import lax
from jax.experimental import pallas as pl
from jax.experimental.pallas import tpu as pltpu
from jax.experimental.pallas import tpu_sc as plsc

VOCAB = 1000000
NUM_WAVES = 16
D_MODEL = 64
B, T = 4096, 50
N = B * T
CH = 128

_info = plsc.get_sparse_core_info()
NC, NS = _info.num_cores, _info.num_subcores
NW = NC * NS
PER_W = N // NW
CHUNKS = PER_W // CH

TWO_PI = float(2.0 * math.pi)


CROWS = CH * NUM_WAVES // 128


def _sc_gather():
    mesh = plsc.VectorSubcoreMesh(core_axis_name="c", subcore_axis_name="s")
    out = jax.ShapeDtypeStruct((N // 8, 128), jnp.float32)
    gbuf = pltpu.VMEM((CH, NUM_WAVES), jnp.float32)

    cbuf = pltpu.VMEM((CROWS, 128), jnp.float32)

    @functools.partial(
        pl.kernel,
        mesh=mesh,
        out_type=[out, out, out],
        compiler_params=pltpu.CompilerParams(use_tc_tiling_on_sc=False),
        scratch_types=[
            pltpu.VMEM((PER_W,), jnp.int32),
            gbuf, gbuf, gbuf,
            cbuf, cbuf, cbuf,
            pltpu.SemaphoreType.DMA,
        ],
    )
    def gather_k(ids_hbm, fr_hbm, ph_hbm, am_hbm,
                 fr_out, ph_out, am_out,
                 idx_v, g0, g1, g2, c0, c1, c2, sem):
        wid = lax.axis_index("s") * NC + lax.axis_index("c")
        pltpu.sync_copy(ids_hbm.at[pl.ds(wid * PER_W, PER_W)], idx_v)

        def body(j, carry):
            idx = idx_v.at[pl.ds(j * CH, CH)]
            d0 = pltpu.async_copy(fr_hbm.at[idx], g0, sem)
            d1 = pltpu.async_copy(ph_hbm.at[idx], g1, sem)
            d2 = pltpu.async_copy(am_hbm.at[idx], g2, sem)
            d0.wait()
            d1.wait()
            d2.wait()

            def row_body(i, _):
                for k in range(8):
                    t = i * 8 + k
                    c0[i, pl.ds(k * 16, 16)] = g0[t, :]
                    c1[i, pl.ds(k * 16, 16)] = g1[t, :]
                    c2[i, pl.ds(k * 16, 16)] = g2[t, :]
                return 0

            lax.fori_loop(0, CROWS, row_body, 0)

            row = wid * (PER_W // 8) + j * CROWS
            pltpu.sync_copy(c0, fr_out.at[pl.ds(row, CROWS)])
            pltpu.sync_copy(c1, ph_out.at[pl.ds(row, CROWS)])
            pltpu.sync_copy(c2, am_out.at[pl.ds(row, CROWS)])
            return carry

        lax.fori_loop(0, CHUNKS, body, 0)

    return gather_k


_gather = _sc_gather()

BB = 800
ROWS = N // 8
GRID = ROWS // BB


def _tc_body(fr_ref, ph_ref, am_ref, pp_ref, wb_ref, bb_ref, out_ref):
    pp = jnp.tile(pp_ref[...], (BB // 25, 1))
    wp = fr_ref[...] * TWO_PI + ph_ref[...] + pp
    am = am_ref[...]
    sw = am * jnp.sin(wp)
    cw = am * jnp.cos(wp)
    x = jnp.concatenate([sw, cw], axis=1)
    y = jnp.dot(x, wb_ref[...], preferred_element_type=jnp.float32,
                precision=lax.Precision.HIGHEST)
    out_ref[...] = y + bb_ref[...]


def _tc_compute(fr, ph, am, pp_table, w_big, b_big):
    blk = lambda i: (i, 0)
    const = lambda i: (0, 0)
    return pl.pallas_call(
        _tc_body,
        grid=(GRID,),
        in_specs=[
            pl.BlockSpec((BB, 128), blk),
            pl.BlockSpec((BB, 128), blk),
            pl.BlockSpec((BB, 128), blk),
            pl.BlockSpec((25, 128), const),
            pl.BlockSpec((256, 512), const),
            pl.BlockSpec((1, 512), const),
        ],
        out_specs=pl.BlockSpec((BB, 512), blk),
        out_shape=jax.ShapeDtypeStruct((ROWS, 512), jnp.float32),
    )(fr, ph, am, pp_table, w_big, b_big)


def kernel(token_ids, token_freqs, token_phases, token_amps, W, b, pos_freq):
    ids1d = token_ids.reshape(N).astype(jnp.int32)
    fr, ph, am = _gather(ids1d, token_freqs, token_phases, token_amps)

    pos = (jnp.arange(200, dtype=jnp.float32) % T).reshape(200, 1)
    pp_table = (pos * pos_freq.reshape(1, NUM_WAVES)).reshape(25, 128)

    eye8 = jnp.eye(8, dtype=jnp.float32)
    w_big = jnp.concatenate(
        [jnp.kron(eye8, W[:NUM_WAVES]), jnp.kron(eye8, W[NUM_WAVES:])], axis=0)
    b_big = jnp.tile(b, 8).reshape(1, 512)

    out = _tc_compute(fr, ph, am, pp_table, w_big, b_big)
    return out.reshape(B, T, D_MODEL)
